# baseline (speedup 1.0000x reference)
import functools

import jax
import jax.numpy as jnp
from jax import lax
from jax.experimental import pallas as pl
from jax.experimental.pallas import tpu as pltpu
from jax.experimental.pallas import tpu_sc as plsc

N = 10000
E = 320000
D = 128
NC, NS = 2, 16
NW = NC * NS
K = 125
ROWS_PER_TILE = E // K // NW
GROUP = 16
NPAD = 10240
NODES_PER_TILE = NPAD // NS
DEG_W = 128

_mesh = plsc.VectorSubcoreMesh(core_axis_name="c", subcore_axis_name="s")


def _deg_body(dst_hbm, ones_hbm, zeros_hbm, out_hbm, idx, ones, acc):
    c = lax.axis_index("c")
    s = lax.axis_index("s")
    wid = c * NS + s
    row0 = s * NODES_PER_TILE
    pltpu.sync_copy(zeros_hbm, acc.at[pl.ds(row0, NODES_PER_TILE)])
    pltpu.sync_copy(ones_hbm, ones)
    pltpu.sync_copy(dst_hbm.at[pl.ds(wid * ROWS_PER_TILE, ROWS_PER_TILE)], idx)
    plsc.subcore_barrier()

    def body(i, carry):
        pltpu.sync_copy(ones, acc.at[idx.at[i]], add=True)
        return carry

    lax.fori_loop(0, ROWS_PER_TILE, body, None)
    plsc.subcore_barrier()
    pltpu.sync_copy(acc.at[pl.ds(row0, NODES_PER_TILE)],
                    out_hbm.at[pl.ds(c * NPAD + row0, NODES_PER_TILE)])


def _msg_body(h_hbm, src_hbm, dst_hbm, zeros_hbm, out_hbm,
              idx_s, idx_d, buf0, buf1, acc, sem0, sem1):
    c = lax.axis_index("c")
    s = lax.axis_index("s")
    wid = c * NS + s
    row0 = s * NODES_PER_TILE
    base = wid * ROWS_PER_TILE

    def load_group(m):
        slot = (m % 2) * GROUP
        pltpu.sync_copy(src_hbm.at[pl.ds(base + m * GROUP, GROUP)],
                        idx_s.at[pl.ds(slot, GROUP)])
        pltpu.sync_copy(dst_hbm.at[pl.ds(base + m * GROUP, GROUP)],
                        idx_d.at[pl.ds(slot, GROUP)])

    def start(i, buf, sem):
        pltpu.async_copy(h_hbm.at[idx_s.at[lax.rem(i, 2 * GROUP)]], buf, sem)

    def finish(i, buf, sem):
        pltpu.make_async_copy(h_hbm.at[idx_s.at[lax.rem(i, 2 * GROUP)]],
                              buf, sem).wait()
        pltpu.sync_copy(buf, acc.at[idx_d.at[lax.rem(i, 2 * GROUP)]], add=True)

    pltpu.sync_copy(zeros_hbm, acc.at[pl.ds(row0, NODES_PER_TILE)])
    load_group(0)
    plsc.subcore_barrier()
    start(0, buf0, sem0)
    start(1, buf1, sem1)

    n_groups = ROWS_PER_TILE // GROUP
    for g in range(n_groups):
        if g < n_groups - 1:
            load_group(g + 1)

        def inner(j2, carry, g=g):
            i0 = g * GROUP + 2 * j2
            finish(i0, buf0, sem0)
            start(i0 + 2, buf0, sem0)
            finish(i0 + 1, buf1, sem1)
            start(i0 + 3, buf1, sem1)
            return carry

        n_inner = GROUP // 2 - (1 if g == n_groups - 1 else 0)
        lax.fori_loop(0, n_inner, inner, None)
    finish(ROWS_PER_TILE - 2, buf0, sem0)
    finish(ROWS_PER_TILE - 1, buf1, sem1)
    plsc.subcore_barrier()
    pltpu.sync_copy(acc.at[pl.ds(row0, NODES_PER_TILE)],
                    out_hbm.at[pl.ds(c * NPAD + row0, NODES_PER_TILE)])


def _build_deg_kernel(interpret=False):
    return pl.kernel(
        _deg_body,
        out_type=jax.ShapeDtypeStruct((2 * NPAD, DEG_W), jnp.float32),
        mesh=_mesh,
        scratch_types=[
            pltpu.VMEM((ROWS_PER_TILE, K), jnp.int32),
            pltpu.VMEM((K, DEG_W), jnp.float32),
            pltpu.VMEM_SHARED((NPAD, DEG_W), jnp.float32),
        ],
        interpret=interpret,
    )


def _build_msg_kernel(interpret=False):
    return pl.kernel(
        _msg_body,
        out_type=jax.ShapeDtypeStruct((2 * NPAD, D), jnp.float32),
        mesh=_mesh,
        scratch_types=[
            pltpu.VMEM((2 * GROUP, K), jnp.int32),
            pltpu.VMEM((2 * GROUP, K), jnp.int32),
            pltpu.VMEM((K, D), jnp.float32),
            pltpu.VMEM((K, D), jnp.float32),
            pltpu.VMEM_SHARED((NPAD, D), jnp.float32),
            pltpu.SemaphoreType.DMA,
            pltpu.SemaphoreType.DMA,
        ],
        interpret=interpret,
    )


_deg_kernel = _build_deg_kernel()
_msg_kernel = _build_msg_kernel()

R = 1000


def _dinv_col(d0_ref, d1_ref):
    deg = d0_ref[:, :1] + d1_ref[:, :1] + 1.0
    return lax.rsqrt(deg)


def _mmu_body(x_ref, w_ref, o_ref):
    o_ref[...] = jnp.dot(x_ref[...], w_ref[...],
                         preferred_element_type=jnp.float32)


def _scale_body(u_ref, d0_ref, d1_ref, o_ref):
    o_ref[...] = u_ref[...] * _dinv_col(d0_ref, d1_ref)


def _mm2_body(p0_ref, p1_ref, h_ref, d0_ref, d1_ref, b_ref, w_ref, o_ref):
    dinv = _dinv_col(d0_ref, d1_ref)
    t = jnp.maximum(dinv * (p0_ref[...] + p1_ref[...] + h_ref[...])
                    + b_ref[...], 0.0)
    o_ref[...] = dinv * jnp.dot(t, w_ref[...], preferred_element_type=jnp.float32)


def _fin_body(q0_ref, q1_ref, h_ref, d0_ref, d1_ref, b_ref, x0_ref, o_ref):
    dinv = _dinv_col(d0_ref, d1_ref)
    x2 = jnp.maximum(dinv * (q0_ref[...] + q1_ref[...] + h_ref[...])
                     + b_ref[...], 0.0)
    o_ref[...] = (x0_ref[...] + x2) * 0.5


_nd_spec = pl.BlockSpec((R, D), lambda i: (i, 0))
_deg_spec = pl.BlockSpec((R, DEG_W), lambda i: (i, 0))
_w_spec = pl.BlockSpec((D, D), lambda i: (0, 0))
_b_spec = pl.BlockSpec((1, D), lambda i: (0, 0))
_out_nd = jax.ShapeDtypeStruct((N, D), jnp.float32)

_mmu = pl.pallas_call(
    _mmu_body, grid=(N // R,),
    in_specs=[_nd_spec, _w_spec],
    out_specs=_nd_spec, out_shape=_out_nd)

_scale = pl.pallas_call(
    _scale_body, grid=(N // R,),
    in_specs=[_nd_spec, _deg_spec, _deg_spec],
    out_specs=_nd_spec, out_shape=_out_nd)

_mm2 = pl.pallas_call(
    _mm2_body, grid=(N // R,),
    in_specs=[_nd_spec, _nd_spec, _nd_spec, _deg_spec, _deg_spec,
              _b_spec, _w_spec],
    out_specs=_nd_spec, out_shape=_out_nd)

_fin = pl.pallas_call(
    _fin_body, grid=(N // R,),
    in_specs=[_nd_spec, _nd_spec, _nd_spec, _deg_spec, _deg_spec,
              _b_spec, _nd_spec],
    out_specs=_nd_spec, out_shape=_out_nd)


def kernel(inputs, edges, W1, b1, W2, b2):
    edges = edges.astype(jnp.int32)
    src = edges[0].reshape(E // K, K)
    dst = edges[1].reshape(E // K, K)
    ones_w = jnp.ones((K, DEG_W), jnp.float32)
    zeros_msg = jnp.zeros((NODES_PER_TILE, D), jnp.float32)

    u1 = _mmu(inputs, W1)
    degp = _deg_kernel(dst, ones_w, zeros_msg)
    d0, d1 = degp[:N], degp[NPAD:NPAD + N]
    h1 = _scale(u1, d0, d1)
    p = _msg_kernel(h1, src, dst, zeros_msg)
    h2 = _mm2(p[:N], p[NPAD:NPAD + N], h1, d0, d1, b1.reshape(1, D), W2)
    q = _msg_kernel(h2, src, dst, zeros_msg)
    out = _fin(q[:N], q[NPAD:NPAD + N], h2, d0, d1, b2.reshape(1, D), inputs)
    return out


# speedup vs baseline: 1.2012x; 1.2012x over previous
import functools

import jax
import jax.numpy as jnp
from jax import lax
from jax.experimental import pallas as pl
from jax.experimental.pallas import tpu as pltpu
from jax.experimental.pallas import tpu_sc as plsc

N = 10000
E = 320000
D = 128
NC, NS = 2, 16
NW = NC * NS
K = 125
ROWS_PER_TILE = E // K // NW
GROUP = 16
NPAD = 10240
NODES_PER_TILE = NPAD // NS

_mesh = plsc.VectorSubcoreMesh(core_axis_name="c", subcore_axis_name="s")


EDGES_PER_TILE = E // NW


def _deg_body(edges_hbm, out_hbm, idxv, degloc, tmp, sums, stage):
    c = lax.axis_index("c")
    s = lax.axis_index("s")
    wid = c * NS + s
    pltpu.sync_copy(edges_hbm.at[pl.ds(E + wid * EDGES_PER_TILE,
                                       EDGES_PER_TILE)], idxv)

    def zero16(i, carry):
        degloc[pl.ds(i * 16, 16)] = jnp.zeros((16,), jnp.float32)
        return carry

    lax.fori_loop(0, NPAD // 16, zero16, None)

    ones = jnp.ones((16,), jnp.float32)

    def count(j, carry):
        vals = idxv[pl.ds(j * 16, 16)]
        plsc.addupdate_scatter(degloc, [vals], ones)
        return carry

    lax.fori_loop(0, EDGES_PER_TILE // 16, count, None)

    pltpu.sync_copy(degloc, stage.at[s])
    plsc.subcore_barrier()
    n0 = s * NODES_PER_TILE
    pltpu.sync_copy(stage.at[:, pl.ds(n0, NODES_PER_TILE)], tmp)

    def sumcol(i, carry):
        acc = tmp[0, pl.ds(i * 16, 16)]
        for w in range(1, NS):
            acc = acc + tmp[w, pl.ds(i * 16, 16)]
        sums[pl.ds(i * 16, 16)] = acc
        return carry

    lax.fori_loop(0, NODES_PER_TILE // 16, sumcol, None)
    pltpu.sync_copy(sums, out_hbm.at[pl.ds(c * NPAD + n0, NODES_PER_TILE)])


def _msg_body(h_hbm, edges_hbm, zeros_hbm, out_hbm,
              idx_s, idx_d, buf0, buf1, acc, sem0, sem1):
    c = lax.axis_index("c")
    s = lax.axis_index("s")
    wid = c * NS + s
    row0 = s * NODES_PER_TILE
    base = wid * ROWS_PER_TILE

    def load_group(m):
        slot = (m % 2) * GROUP
        pltpu.sync_copy(edges_hbm.at[pl.ds(base + m * GROUP, GROUP)],
                        idx_s.at[pl.ds(slot, GROUP)])
        pltpu.sync_copy(edges_hbm.at[pl.ds(E // K + base + m * GROUP, GROUP)],
                        idx_d.at[pl.ds(slot, GROUP)])

    def start(i, buf, sem):
        pltpu.async_copy(h_hbm.at[idx_s.at[lax.rem(i, 2 * GROUP)]], buf, sem)

    def finish(i, buf, sem):
        pltpu.make_async_copy(h_hbm.at[idx_s.at[lax.rem(i, 2 * GROUP)]],
                              buf, sem).wait()
        pltpu.sync_copy(buf, acc.at[idx_d.at[lax.rem(i, 2 * GROUP)]], add=True)

    pltpu.sync_copy(zeros_hbm, acc.at[pl.ds(row0, NODES_PER_TILE)])
    load_group(0)
    plsc.subcore_barrier()
    start(0, buf0, sem0)
    start(1, buf1, sem1)

    n_groups = ROWS_PER_TILE // GROUP
    for g in range(n_groups):
        if g < n_groups - 1:
            load_group(g + 1)

        def inner(j2, carry, g=g):
            i0 = g * GROUP + 2 * j2
            finish(i0, buf0, sem0)
            start(i0 + 2, buf0, sem0)
            finish(i0 + 1, buf1, sem1)
            start(i0 + 3, buf1, sem1)
            return carry

        n_inner = GROUP // 2 - (1 if g == n_groups - 1 else 0)
        lax.fori_loop(0, n_inner, inner, None)
    finish(ROWS_PER_TILE - 2, buf0, sem0)
    finish(ROWS_PER_TILE - 1, buf1, sem1)
    plsc.subcore_barrier()
    pltpu.sync_copy(acc.at[pl.ds(row0, NODES_PER_TILE)],
                    out_hbm.at[pl.ds(c * NPAD + row0, NODES_PER_TILE)])


def _build_deg_kernel(interpret=False):
    return pl.kernel(
        _deg_body,
        out_type=jax.ShapeDtypeStruct((2 * NPAD,), jnp.float32),
        mesh=_mesh,
        scratch_types=[
            pltpu.VMEM((EDGES_PER_TILE,), jnp.int32),
            pltpu.VMEM((NPAD,), jnp.float32),
            pltpu.VMEM((NS, NODES_PER_TILE), jnp.float32),
            pltpu.VMEM((NODES_PER_TILE,), jnp.float32),
            pltpu.VMEM_SHARED((NS, NPAD), jnp.float32),
        ],
        compiler_params=pltpu.CompilerParams(needs_layout_passes=False),
        interpret=interpret,
    )


def _build_msg_kernel(interpret=False):
    return pl.kernel(
        _msg_body,
        out_type=jax.ShapeDtypeStruct((2 * NPAD, D), jnp.float32),
        mesh=_mesh,
        scratch_types=[
            pltpu.VMEM((2 * GROUP, K), jnp.int32),
            pltpu.VMEM((2 * GROUP, K), jnp.int32),
            pltpu.VMEM((K, D), jnp.float32),
            pltpu.VMEM((K, D), jnp.float32),
            pltpu.VMEM_SHARED((NPAD, D), jnp.float32),
            pltpu.SemaphoreType.DMA,
            pltpu.SemaphoreType.DMA,
        ],
        interpret=interpret,
    )


_deg_kernel = _build_deg_kernel()
_msg_kernel = _build_msg_kernel()

R = 1000


def _dinv_col(d0_ref, d1_ref):
    deg = d0_ref[...] + d1_ref[...] + 1.0
    return lax.rsqrt(deg)


def _mmu_body(x_ref, w_ref, o_ref):
    o_ref[...] = jnp.dot(x_ref[...], w_ref[...],
                         preferred_element_type=jnp.float32)


def _scale_body(u_ref, d0_ref, d1_ref, o_ref):
    o_ref[...] = u_ref[...] * _dinv_col(d0_ref, d1_ref)


def _mm2_body(p0_ref, p1_ref, h_ref, d0_ref, d1_ref, b_ref, w_ref, o_ref):
    dinv = _dinv_col(d0_ref, d1_ref)
    t = jnp.maximum(dinv * (p0_ref[...] + p1_ref[...] + h_ref[...])
                    + b_ref[...], 0.0)
    o_ref[...] = dinv * jnp.dot(t, w_ref[...], preferred_element_type=jnp.float32)


def _fin_body(q0_ref, q1_ref, h_ref, d0_ref, d1_ref, b_ref, x0_ref, o_ref):
    dinv = _dinv_col(d0_ref, d1_ref)
    x2 = jnp.maximum(dinv * (q0_ref[...] + q1_ref[...] + h_ref[...])
                     + b_ref[...], 0.0)
    o_ref[...] = (x0_ref[...] + x2) * 0.5


_nd_spec = pl.BlockSpec((R, D), lambda i: (i, 0))
_deg_spec = pl.BlockSpec((R, 1), lambda i: (i, 0))
_w_spec = pl.BlockSpec((D, D), lambda i: (0, 0))
_b_spec = pl.BlockSpec((1, D), lambda i: (0, 0))
_out_nd = jax.ShapeDtypeStruct((N, D), jnp.float32)

_mmu = pl.pallas_call(
    _mmu_body, grid=(N // R,),
    in_specs=[_nd_spec, _w_spec],
    out_specs=_nd_spec, out_shape=_out_nd)

_scale = pl.pallas_call(
    _scale_body, grid=(N // R,),
    in_specs=[_nd_spec, _deg_spec, _deg_spec],
    out_specs=_nd_spec, out_shape=_out_nd)

_mm2 = pl.pallas_call(
    _mm2_body, grid=(N // R,),
    in_specs=[_nd_spec, _nd_spec, _nd_spec, _deg_spec, _deg_spec,
              _b_spec, _w_spec],
    out_specs=_nd_spec, out_shape=_out_nd)

_fin = pl.pallas_call(
    _fin_body, grid=(N // R,),
    in_specs=[_nd_spec, _nd_spec, _nd_spec, _deg_spec, _deg_spec,
              _b_spec, _nd_spec],
    out_specs=_nd_spec, out_shape=_out_nd)


def kernel(inputs, edges, W1, b1, W2, b2):
    edges2 = edges.astype(jnp.int32).reshape(2 * (E // K), K)
    zeros_msg = jnp.zeros((NODES_PER_TILE, D), jnp.float32)

    u1 = _mmu(inputs, W1)
    degp = _deg_kernel(edges2.reshape(-1))
    d0 = degp[:N].reshape(N, 1)
    d1 = degp[NPAD:NPAD + N].reshape(N, 1)
    h1 = _scale(u1, d0, d1)
    p = _msg_kernel(h1, edges2, zeros_msg)
    h2 = _mm2(p[:N], p[NPAD:NPAD + N], h1, d0, d1, b1.reshape(1, D), W2)
    q = _msg_kernel(h2, edges2, zeros_msg)
    out = _fin(q[:N], q[NPAD:NPAD + N], h2, d0, d1, b2.reshape(1, D), inputs)
    return out


# speedup vs baseline: 1.2017x; 1.0004x over previous
import functools

import jax
import jax.numpy as jnp
from jax import lax
from jax.experimental import pallas as pl
from jax.experimental.pallas import tpu as pltpu
from jax.experimental.pallas import tpu_sc as plsc

N = 10000
E = 320000
D = 128
NC, NS = 2, 16
NW = NC * NS
K = 125
ROWS_PER_TILE = E // K // NW
GROUP = 16
NPAD = 10240
NODES_PER_TILE = NPAD // NS

_mesh = plsc.VectorSubcoreMesh(core_axis_name="c", subcore_axis_name="s")


EDGES_PER_TILE = E // NW


def _deg_body(edges_hbm, out_hbm, idxv, degloc, tmp, sums, stage):
    c = lax.axis_index("c")
    s = lax.axis_index("s")
    wid = c * NS + s
    pltpu.sync_copy(
        edges_hbm.at[pl.ds(E // K + wid * ROWS_PER_TILE, ROWS_PER_TILE)],
        idxv)

    def zero16(i, carry):
        degloc[pl.ds(i * 16, 16)] = jnp.zeros((16,), jnp.float32)
        return carry

    lax.fori_loop(0, NPAD // 16, zero16, None)

    ones = jnp.ones((16,), jnp.float32)
    tail_mask = lax.iota(jnp.int32, 16) >= jnp.full((16,), (16 - K % 16) % 16,
                                                    jnp.int32)

    def count_row(r, carry):
        for g in range(pl.cdiv(K, 16)):
            last = g == pl.cdiv(K, 16) - 1 and K % 16 != 0
            off = K - 16 if last else g * 16
            vals = idxv[r, pl.ds(off, 16)]
            plsc.addupdate_scatter(degloc, [vals], ones,
                                   mask=tail_mask if last else None)
        return carry

    lax.fori_loop(0, ROWS_PER_TILE, count_row, None)

    pltpu.sync_copy(degloc, stage.at[s])
    plsc.subcore_barrier()
    n0 = s * NODES_PER_TILE
    pltpu.sync_copy(stage.at[:, pl.ds(n0, NODES_PER_TILE)], tmp)

    def sumcol(i, carry):
        acc = tmp[0, pl.ds(i * 16, 16)]
        for w in range(1, NS):
            acc = acc + tmp[w, pl.ds(i * 16, 16)]
        sums[pl.ds(i * 16, 16)] = acc
        return carry

    lax.fori_loop(0, NODES_PER_TILE // 16, sumcol, None)
    pltpu.sync_copy(sums, out_hbm.at[pl.ds(c * NPAD + n0, NODES_PER_TILE)])


def _msg_body(h_hbm, edges_hbm, zeros_hbm, out_hbm,
              idx_s, idx_d, buf0, buf1, acc, sem0, sem1):
    c = lax.axis_index("c")
    s = lax.axis_index("s")
    wid = c * NS + s
    row0 = s * NODES_PER_TILE
    base = wid * ROWS_PER_TILE

    def load_group(m):
        slot = (m % 2) * GROUP
        pltpu.sync_copy(edges_hbm.at[pl.ds(base + m * GROUP, GROUP)],
                        idx_s.at[pl.ds(slot, GROUP)])
        pltpu.sync_copy(edges_hbm.at[pl.ds(E // K + base + m * GROUP, GROUP)],
                        idx_d.at[pl.ds(slot, GROUP)])

    def start(i, buf, sem):
        pltpu.async_copy(h_hbm.at[idx_s.at[lax.rem(i, 2 * GROUP)]], buf, sem)

    def finish(i, buf, sem):
        pltpu.make_async_copy(h_hbm.at[idx_s.at[lax.rem(i, 2 * GROUP)]],
                              buf, sem).wait()
        pltpu.sync_copy(buf, acc.at[idx_d.at[lax.rem(i, 2 * GROUP)]], add=True)

    pltpu.sync_copy(zeros_hbm, acc.at[pl.ds(row0, NODES_PER_TILE)])
    load_group(0)
    plsc.subcore_barrier()
    start(0, buf0, sem0)
    start(1, buf1, sem1)

    n_groups = ROWS_PER_TILE // GROUP
    for g in range(n_groups):
        if g < n_groups - 1:
            load_group(g + 1)

        def inner(j2, carry, g=g):
            i0 = g * GROUP + 2 * j2
            finish(i0, buf0, sem0)
            start(i0 + 2, buf0, sem0)
            finish(i0 + 1, buf1, sem1)
            start(i0 + 3, buf1, sem1)
            return carry

        n_inner = GROUP // 2 - (1 if g == n_groups - 1 else 0)
        lax.fori_loop(0, n_inner, inner, None)
    finish(ROWS_PER_TILE - 2, buf0, sem0)
    finish(ROWS_PER_TILE - 1, buf1, sem1)
    plsc.subcore_barrier()
    pltpu.sync_copy(acc.at[pl.ds(row0, NODES_PER_TILE)],
                    out_hbm.at[pl.ds(c * NPAD + row0, NODES_PER_TILE)])


def _build_deg_kernel(interpret=False):
    return pl.kernel(
        _deg_body,
        out_type=jax.ShapeDtypeStruct((2 * NPAD,), jnp.float32),
        mesh=_mesh,
        scratch_types=[
            pltpu.VMEM((ROWS_PER_TILE, K), jnp.int32),
            pltpu.VMEM((NPAD,), jnp.float32),
            pltpu.VMEM((NS, NODES_PER_TILE), jnp.float32),
            pltpu.VMEM((NODES_PER_TILE,), jnp.float32),
            pltpu.VMEM_SHARED((NS, NPAD), jnp.float32),
        ],
        compiler_params=pltpu.CompilerParams(needs_layout_passes=False),
        interpret=interpret,
    )


def _build_msg_kernel(interpret=False):
    return pl.kernel(
        _msg_body,
        out_type=jax.ShapeDtypeStruct((2 * NPAD, D), jnp.float32),
        mesh=_mesh,
        scratch_types=[
            pltpu.VMEM((2 * GROUP, K), jnp.int32),
            pltpu.VMEM((2 * GROUP, K), jnp.int32),
            pltpu.VMEM((K, D), jnp.float32),
            pltpu.VMEM((K, D), jnp.float32),
            pltpu.VMEM_SHARED((NPAD, D), jnp.float32),
            pltpu.SemaphoreType.DMA,
            pltpu.SemaphoreType.DMA,
        ],
        interpret=interpret,
    )


_deg_kernel = _build_deg_kernel()
_msg_kernel = _build_msg_kernel()

R = 1000


def _dinv_col(d0_ref, d1_ref):
    deg = d0_ref[...] + d1_ref[...] + 1.0
    return lax.rsqrt(deg)


def _mm1_body(x_ref, w_ref, d0_ref, d1_ref, o_ref):
    h = jnp.dot(x_ref[...], w_ref[...], preferred_element_type=jnp.float32)
    o_ref[...] = h * _dinv_col(d0_ref, d1_ref)


def _mm2_body(p0_ref, p1_ref, h_ref, d0_ref, d1_ref, b_ref, w_ref, o_ref):
    dinv = _dinv_col(d0_ref, d1_ref)
    t = jnp.maximum(dinv * (p0_ref[...] + p1_ref[...] + h_ref[...])
                    + b_ref[...], 0.0)
    o_ref[...] = dinv * jnp.dot(t, w_ref[...], preferred_element_type=jnp.float32)


def _fin_body(q0_ref, q1_ref, h_ref, d0_ref, d1_ref, b_ref, x0_ref, o_ref):
    dinv = _dinv_col(d0_ref, d1_ref)
    x2 = jnp.maximum(dinv * (q0_ref[...] + q1_ref[...] + h_ref[...])
                     + b_ref[...], 0.0)
    o_ref[...] = (x0_ref[...] + x2) * 0.5


_nd_spec = pl.BlockSpec((R, D), lambda i: (i, 0))
_deg_spec = pl.BlockSpec((R, 1), lambda i: (i, 0))
_w_spec = pl.BlockSpec((D, D), lambda i: (0, 0))
_b_spec = pl.BlockSpec((1, D), lambda i: (0, 0))
_out_nd = jax.ShapeDtypeStruct((N, D), jnp.float32)

_mm1 = pl.pallas_call(
    _mm1_body, grid=(N // R,),
    in_specs=[_nd_spec, _w_spec, _deg_spec, _deg_spec],
    out_specs=_nd_spec, out_shape=_out_nd)

_mm2 = pl.pallas_call(
    _mm2_body, grid=(N // R,),
    in_specs=[_nd_spec, _nd_spec, _nd_spec, _deg_spec, _deg_spec,
              _b_spec, _w_spec],
    out_specs=_nd_spec, out_shape=_out_nd)

_fin = pl.pallas_call(
    _fin_body, grid=(N // R,),
    in_specs=[_nd_spec, _nd_spec, _nd_spec, _deg_spec, _deg_spec,
              _b_spec, _nd_spec],
    out_specs=_nd_spec, out_shape=_out_nd)


def kernel(inputs, edges, W1, b1, W2, b2):
    edges2 = edges.astype(jnp.int32).reshape(2 * (E // K), K)
    zeros_msg = jnp.zeros((NODES_PER_TILE, D), jnp.float32)

    degp = _deg_kernel(edges2)
    d0 = degp[:N].reshape(N, 1)
    d1 = degp[NPAD:NPAD + N].reshape(N, 1)
    h1 = _mm1(inputs, W1, d0, d1)
    p = _msg_kernel(h1, edges2, zeros_msg)
    h2 = _mm2(p[:N], p[NPAD:NPAD + N], h1, d0, d1, b1.reshape(1, D), W2)
    q = _msg_kernel(h2, edges2, zeros_msg)
    out = _fin(q[:N], q[NPAD:NPAD + N], h2, d0, d1, b2.reshape(1, D), inputs)
    return out


# speedup vs baseline: 1.2353x; 1.0280x over previous
import functools

import jax
import jax.numpy as jnp
from jax import lax
from jax.experimental import pallas as pl
from jax.experimental.pallas import tpu as pltpu
from jax.experimental.pallas import tpu_sc as plsc

N = 10000
E = 320000
D = 128
NC, NS = 2, 16
NW = NC * NS
K = 125
ROWS_PER_TILE = E // K // NW
GROUP = 16
NPAD = 10240
NODES_PER_TILE = NPAD // NS

_mesh = plsc.VectorSubcoreMesh(core_axis_name="c", subcore_axis_name="s")


EDGES_PER_TILE = E // NW


def _deg_body(edges_hbm, out_hbm, idxv, degloc, tmp, sums, stage):
    c = lax.axis_index("c")
    s = lax.axis_index("s")
    wid = c * NS + s
    pltpu.sync_copy(
        edges_hbm.at[pl.ds(E // K + wid * ROWS_PER_TILE, ROWS_PER_TILE)],
        idxv)

    def zero16(i, carry):
        degloc[pl.ds(i * 16, 16)] = jnp.zeros((16,), jnp.float32)
        return carry

    lax.fori_loop(0, NPAD // 16, zero16, None)

    ones = jnp.ones((16,), jnp.float32)
    tail_mask = lax.iota(jnp.int32, 16) >= jnp.full((16,), (16 - K % 16) % 16,
                                                    jnp.int32)

    def count_row(r, carry):
        for g in range(pl.cdiv(K, 16)):
            last = g == pl.cdiv(K, 16) - 1 and K % 16 != 0
            off = K - 16 if last else g * 16
            vals = idxv[r, pl.ds(off, 16)]
            plsc.addupdate_scatter(degloc, [vals], ones,
                                   mask=tail_mask if last else None)
        return carry

    lax.fori_loop(0, ROWS_PER_TILE, count_row, None)

    pltpu.sync_copy(degloc, stage.at[s])
    plsc.subcore_barrier()
    n0 = s * NODES_PER_TILE
    pltpu.sync_copy(stage.at[:, pl.ds(n0, NODES_PER_TILE)], tmp)

    def sumcol(i, carry):
        acc = tmp[0, pl.ds(i * 16, 16)]
        for w in range(1, NS):
            acc = acc + tmp[w, pl.ds(i * 16, 16)]
        sums[pl.ds(i * 16, 16)] = acc
        return carry

    lax.fori_loop(0, NODES_PER_TILE // 16, sumcol, None)
    pltpu.sync_copy(sums, out_hbm.at[pl.ds(c * NPAD + n0, NODES_PER_TILE)])


def _msg_body(h_hbm, edges_hbm, zeros_hbm, out_hbm,
              idx_s, idx_d, buf0, buf1, acc, sem0, sem1):
    c = lax.axis_index("c")
    s = lax.axis_index("s")
    wid = c * NS + s
    row0 = s * NODES_PER_TILE
    base = wid * ROWS_PER_TILE

    def load_group(m):
        slot = (m % 2) * GROUP
        pltpu.sync_copy(edges_hbm.at[pl.ds(base + m * GROUP, GROUP)],
                        idx_s.at[pl.ds(slot, GROUP)])
        pltpu.sync_copy(edges_hbm.at[pl.ds(E // K + base + m * GROUP, GROUP)],
                        idx_d.at[pl.ds(slot, GROUP)])

    def start(i, buf, sem):
        pltpu.async_copy(h_hbm.at[idx_s.at[lax.rem(i, 2 * GROUP)]], buf, sem)

    def finish(i, buf, sem):
        pltpu.make_async_copy(h_hbm.at[idx_s.at[lax.rem(i, 2 * GROUP)]],
                              buf, sem).wait()
        pltpu.sync_copy(buf, acc.at[idx_d.at[lax.rem(i, 2 * GROUP)]], add=True)

    pltpu.sync_copy(zeros_hbm, acc.at[pl.ds(row0, NODES_PER_TILE)])
    load_group(0)
    plsc.subcore_barrier()
    start(0, buf0, sem0)
    start(1, buf1, sem1)

    n_groups = ROWS_PER_TILE // GROUP
    for g in range(n_groups):
        if g < n_groups - 1:
            load_group(g + 1)

        def inner(j2, carry, g=g):
            i0 = g * GROUP + 2 * j2
            finish(i0, buf0, sem0)
            start(i0 + 2, buf0, sem0)
            finish(i0 + 1, buf1, sem1)
            start(i0 + 3, buf1, sem1)
            return carry

        n_inner = GROUP // 2 - (1 if g == n_groups - 1 else 0)
        lax.fori_loop(0, n_inner, inner, None)
    finish(ROWS_PER_TILE - 2, buf0, sem0)
    finish(ROWS_PER_TILE - 1, buf1, sem1)
    plsc.subcore_barrier()
    pltpu.sync_copy(acc.at[pl.ds(row0, NODES_PER_TILE)],
                    out_hbm.at[pl.ds(c * NPAD + row0, NODES_PER_TILE)])


def _build_deg_kernel(interpret=False):
    return pl.kernel(
        _deg_body,
        out_type=jax.ShapeDtypeStruct((2 * NPAD,), jnp.float32),
        mesh=_mesh,
        scratch_types=[
            pltpu.VMEM((ROWS_PER_TILE, K), jnp.int32),
            pltpu.VMEM((NPAD,), jnp.float32),
            pltpu.VMEM((NS, NODES_PER_TILE), jnp.float32),
            pltpu.VMEM((NODES_PER_TILE,), jnp.float32),
            pltpu.VMEM_SHARED((NS, NPAD), jnp.float32),
        ],
        compiler_params=pltpu.CompilerParams(needs_layout_passes=False),
        interpret=interpret,
    )


def _build_msg_kernel(interpret=False):
    return pl.kernel(
        _msg_body,
        out_type=jax.ShapeDtypeStruct((2 * NPAD, D), jnp.float32),
        mesh=_mesh,
        scratch_types=[
            pltpu.VMEM((2 * GROUP, K), jnp.int32),
            pltpu.VMEM((2 * GROUP, K), jnp.int32),
            pltpu.VMEM((K, D), jnp.float32),
            pltpu.VMEM((K, D), jnp.float32),
            pltpu.VMEM_SHARED((NPAD, D), jnp.float32),
            pltpu.SemaphoreType.DMA,
            pltpu.SemaphoreType.DMA,
        ],
        interpret=interpret,
    )


_deg_kernel = _build_deg_kernel()
_msg_kernel = _build_msg_kernel()

R = 1000


def _dinv_col(d_ref):
    return lax.rsqrt(d_ref[...] + 1.0)


def _mm1_body(x_ref, w_ref, d_ref, o_ref):
    h = jnp.dot(x_ref[...], w_ref[...], preferred_element_type=jnp.float32)
    o_ref[...] = h * _dinv_col(d_ref)


def _mm2_body(p0_ref, p1_ref, h_ref, d_ref, b_ref, w_ref, o_ref):
    dinv = _dinv_col(d_ref)
    t = jnp.maximum(dinv * (p0_ref[...] + p1_ref[...] + h_ref[...])
                    + b_ref[...], 0.0)
    o_ref[...] = dinv * jnp.dot(t, w_ref[...], preferred_element_type=jnp.float32)


def _fin_body(q0_ref, q1_ref, h_ref, d_ref, b_ref, x0_ref, o_ref):
    dinv = _dinv_col(d_ref)
    x2 = jnp.maximum(dinv * (q0_ref[...] + q1_ref[...] + h_ref[...])
                     + b_ref[...], 0.0)
    o_ref[...] = (x0_ref[...] + x2) * 0.5


_nd_spec = pl.BlockSpec((R, D), lambda i: (i, 0))
_deg_spec = pl.BlockSpec((R, 1), lambda i: (i, 0))
_w_spec = pl.BlockSpec((D, D), lambda i: (0, 0))
_b_spec = pl.BlockSpec((1, D), lambda i: (0, 0))
_out_nd = jax.ShapeDtypeStruct((N, D), jnp.float32)

_mm1 = pl.pallas_call(
    _mm1_body, grid=(N // R,),
    in_specs=[_nd_spec, _w_spec, _deg_spec],
    out_specs=_nd_spec, out_shape=_out_nd)

_mm2 = pl.pallas_call(
    _mm2_body, grid=(N // R,),
    in_specs=[_nd_spec, _nd_spec, _nd_spec, _deg_spec,
              _b_spec, _w_spec],
    out_specs=_nd_spec, out_shape=_out_nd)

_fin = pl.pallas_call(
    _fin_body, grid=(N // R,),
    in_specs=[_nd_spec, _nd_spec, _nd_spec, _deg_spec,
              _b_spec, _nd_spec],
    out_specs=_nd_spec, out_shape=_out_nd)


def kernel(inputs, edges, W1, b1, W2, b2):
    edges2 = edges.astype(jnp.int32).reshape(2 * (E // K), K)
    zeros_msg = jnp.zeros((NODES_PER_TILE, D), jnp.float32)

    degp = _deg_kernel(edges2)
    dsum = (degp[:N] + degp[NPAD:NPAD + N]).reshape(N, 1)
    h1 = _mm1(inputs, W1, dsum)
    p = _msg_kernel(h1, edges2, zeros_msg)
    h2 = _mm2(p[:N], p[NPAD:NPAD + N], h1, dsum, b1.reshape(1, D), W2)
    q = _msg_kernel(h2, edges2, zeros_msg)
    out = _fin(q[:N], q[NPAD:NPAD + N], h2, dsum, b2.reshape(1, D), inputs)
    return out


# speedup vs baseline: 1.2825x; 1.0382x over previous
import functools

import jax
import jax.numpy as jnp
from jax import lax
from jax.experimental import pallas as pl
from jax.experimental.pallas import tpu as pltpu
from jax.experimental.pallas import tpu_sc as plsc

N = 10000
E = 320000
D = 128
NC, NS = 2, 16
NW = NC * NS
K = 125
ROWS_PER_TILE = E // K // NW
GROUP = 16
NPAD = 10240
NODES_PER_TILE = NPAD // NS

_mesh = plsc.VectorSubcoreMesh(core_axis_name="c", subcore_axis_name="s")


EDGES_PER_TILE = E // NW


def _deg_body(edges_hbm, out_hbm, idxv, degloc, tmp, sums, stage):
    c = lax.axis_index("c")
    s = lax.axis_index("s")
    wid = c * NS + s
    pltpu.sync_copy(
        edges_hbm.at[pl.ds(E // K + wid * ROWS_PER_TILE, ROWS_PER_TILE)],
        idxv)

    def zero16(i, carry):
        degloc[pl.ds(i * 16, 16)] = jnp.zeros((16,), jnp.float32)
        return carry

    lax.fori_loop(0, NPAD // 16, zero16, None)

    ones = jnp.ones((16,), jnp.float32)
    tail_mask = lax.iota(jnp.int32, 16) >= jnp.full((16,), (16 - K % 16) % 16,
                                                    jnp.int32)

    def count_row(r, carry):
        for g in range(pl.cdiv(K, 16)):
            last = g == pl.cdiv(K, 16) - 1 and K % 16 != 0
            off = K - 16 if last else g * 16
            vals = idxv[r, pl.ds(off, 16)]
            plsc.addupdate_scatter(degloc, [vals], ones,
                                   mask=tail_mask if last else None)
        return carry

    lax.fori_loop(0, ROWS_PER_TILE, count_row, None)

    pltpu.sync_copy(degloc, stage.at[s])
    plsc.subcore_barrier()
    n0 = s * NODES_PER_TILE
    pltpu.sync_copy(stage.at[:, pl.ds(n0, NODES_PER_TILE)], tmp)

    def sumcol(i, carry):
        acc = tmp[0, pl.ds(i * 16, 16)]
        for w in range(1, NS):
            acc = acc + tmp[w, pl.ds(i * 16, 16)]
        sums[pl.ds(i * 16, 16)] = acc
        return carry

    lax.fori_loop(0, NODES_PER_TILE // 16, sumcol, None)
    pltpu.sync_copy(sums, out_hbm.at[pl.ds(c * NPAD + n0, NODES_PER_TILE)])


def _msg_body(h_hbm, edges_hbm, out_hbm,
              idx_s, idx_d, buf0, buf1, acc, sem0, sem1):
    c = lax.axis_index("c")
    s = lax.axis_index("s")
    wid = c * NS + s
    row0 = s * NODES_PER_TILE
    base = wid * ROWS_PER_TILE

    def load_group(m):
        slot = (m % 2) * GROUP
        pltpu.sync_copy(edges_hbm.at[pl.ds(base + m * GROUP, GROUP)],
                        idx_s.at[pl.ds(slot, GROUP)])
        pltpu.sync_copy(edges_hbm.at[pl.ds(E // K + base + m * GROUP, GROUP)],
                        idx_d.at[pl.ds(slot, GROUP)])

    def start(i, buf, sem):
        pltpu.async_copy(h_hbm.at[idx_s.at[lax.rem(i, 2 * GROUP)]], buf, sem)

    def finish(i, buf, sem):
        pltpu.make_async_copy(h_hbm.at[idx_s.at[lax.rem(i, 2 * GROUP)]],
                              buf, sem).wait()
        pltpu.sync_copy(buf, acc.at[idx_d.at[lax.rem(i, 2 * GROUP)]], add=True)

    def zrow(r, carry):
        for j in range(D // 16):
            buf0[r, pl.ds(j * 16, 16)] = jnp.zeros((16,), jnp.float32)
        return carry

    lax.fori_loop(0, 80, zrow, None)
    for j in range(NODES_PER_TILE // 80):
        pltpu.sync_copy(buf0.at[pl.ds(0, 80)],
                        acc.at[pl.ds(row0 + j * 80, 80)])
    load_group(0)
    plsc.subcore_barrier()
    start(0, buf0, sem0)
    start(1, buf1, sem1)

    n_groups = ROWS_PER_TILE // GROUP
    for g in range(n_groups):
        if g < n_groups - 1:
            load_group(g + 1)

        def inner(j2, carry, g=g):
            i0 = g * GROUP + 2 * j2
            finish(i0, buf0, sem0)
            start(i0 + 2, buf0, sem0)
            finish(i0 + 1, buf1, sem1)
            start(i0 + 3, buf1, sem1)
            return carry

        n_inner = GROUP // 2 - (1 if g == n_groups - 1 else 0)
        lax.fori_loop(0, n_inner, inner, None)
    finish(ROWS_PER_TILE - 2, buf0, sem0)
    finish(ROWS_PER_TILE - 1, buf1, sem1)
    plsc.subcore_barrier()
    pltpu.sync_copy(acc.at[pl.ds(row0, NODES_PER_TILE)],
                    out_hbm.at[pl.ds(c * NPAD + row0, NODES_PER_TILE)])


def _build_deg_kernel(interpret=False):
    return pl.kernel(
        _deg_body,
        out_type=jax.ShapeDtypeStruct((2 * NPAD,), jnp.float32),
        mesh=_mesh,
        scratch_types=[
            pltpu.VMEM((ROWS_PER_TILE, K), jnp.int32),
            pltpu.VMEM((NPAD,), jnp.float32),
            pltpu.VMEM((NS, NODES_PER_TILE), jnp.float32),
            pltpu.VMEM((NODES_PER_TILE,), jnp.float32),
            pltpu.VMEM_SHARED((NS, NPAD), jnp.float32),
        ],
        compiler_params=pltpu.CompilerParams(needs_layout_passes=False),
        interpret=interpret,
    )


def _build_msg_kernel(interpret=False):
    return pl.kernel(
        _msg_body,
        out_type=jax.ShapeDtypeStruct((2 * NPAD, D), jnp.float32),
        mesh=_mesh,
        scratch_types=[
            pltpu.VMEM((2 * GROUP, K), jnp.int32),
            pltpu.VMEM((2 * GROUP, K), jnp.int32),
            pltpu.VMEM((K, D), jnp.float32),
            pltpu.VMEM((K, D), jnp.float32),
            pltpu.VMEM_SHARED((NPAD, D), jnp.float32),
            pltpu.SemaphoreType.DMA,
            pltpu.SemaphoreType.DMA,
        ],
        interpret=interpret,
    )


_deg_kernel = _build_deg_kernel()
_msg_kernel = _build_msg_kernel()

R = 1000


def _dinv_col(d_ref):
    return lax.rsqrt(d_ref[...] + 1.0)


def _mm1_body(x_ref, w_ref, d_ref, o_ref):
    h = jnp.dot(x_ref[...], w_ref[...], preferred_element_type=jnp.float32)
    o_ref[...] = h * _dinv_col(d_ref)


def _mm2_body(p0_ref, p1_ref, h_ref, d_ref, b_ref, w_ref, o_ref):
    dinv = _dinv_col(d_ref)
    t = jnp.maximum(dinv * (p0_ref[...] + p1_ref[...] + h_ref[...])
                    + b_ref[...], 0.0)
    o_ref[...] = dinv * jnp.dot(t, w_ref[...], preferred_element_type=jnp.float32)


def _fin_body(q0_ref, q1_ref, h_ref, d_ref, b_ref, x0_ref, o_ref):
    dinv = _dinv_col(d_ref)
    x2 = jnp.maximum(dinv * (q0_ref[...] + q1_ref[...] + h_ref[...])
                     + b_ref[...], 0.0)
    o_ref[...] = (x0_ref[...] + x2) * 0.5


_nd_spec = pl.BlockSpec((R, D), lambda i: (i, 0))
_deg_spec = pl.BlockSpec((R, 1), lambda i: (i, 0))
_w_spec = pl.BlockSpec((D, D), lambda i: (0, 0))
_b_spec = pl.BlockSpec((1, D), lambda i: (0, 0))
_out_nd = jax.ShapeDtypeStruct((N, D), jnp.float32)

_mm1 = pl.pallas_call(
    _mm1_body, grid=(N // R,),
    in_specs=[_nd_spec, _w_spec, _deg_spec],
    out_specs=_nd_spec, out_shape=_out_nd)

_mm2 = pl.pallas_call(
    _mm2_body, grid=(N // R,),
    in_specs=[_nd_spec, _nd_spec, _nd_spec, _deg_spec,
              _b_spec, _w_spec],
    out_specs=_nd_spec, out_shape=_out_nd)

_fin = pl.pallas_call(
    _fin_body, grid=(N // R,),
    in_specs=[_nd_spec, _nd_spec, _nd_spec, _deg_spec,
              _b_spec, _nd_spec],
    out_specs=_nd_spec, out_shape=_out_nd)


def kernel(inputs, edges, W1, b1, W2, b2):
    edges2 = edges.astype(jnp.int32).reshape(2 * (E // K), K)
    degp = _deg_kernel(edges2)
    dsum = (degp[:N] + degp[NPAD:NPAD + N]).reshape(N, 1)
    h1 = _mm1(inputs, W1, dsum)
    p = _msg_kernel(h1, edges2)
    h2 = _mm2(p[:N], p[NPAD:NPAD + N], h1, dsum, b1.reshape(1, D), W2)
    q = _msg_kernel(h2, edges2)
    out = _fin(q[:N], q[NPAD:NPAD + N], h2, dsum, b2.reshape(1, D), inputs)
    return out
